# baseline (speedup 1.0000x reference)
import numpy as np
import jax
import jax.numpy as jnp
from jax.experimental import pallas as pl
from jax.experimental.pallas import tpu as pltpu

_B = 4
_H_IMG = 112
_H = _W = 28
_C_EMB = 448
_N_BANK = 100
_K_IM = 50
_WIN = 3
_HIGH = jax.lax.Precision.HIGHEST


def _mm_relu_kernel(a_ref, w_ref, o_ref):
    o_ref[...] = jax.nn.relu(
        jax.lax.dot_general(a_ref[...], w_ref[...], (((1,), (0,)), ((), ())),
                            precision=_HIGH,
                            preferred_element_type=jnp.float32))


def _mm_relu(a, w):
    m, _ = a.shape
    _, n = w.shape
    return pl.pallas_call(
        _mm_relu_kernel,
        out_shape=jax.ShapeDtypeStruct((m, n), jnp.float32),
    )(a, w)


def _pool3_kernel(x1_ref, x2_ref, x3_ref, o1_ref, o2_ref, o3_ref):
    for xr, orf in ((x1_ref, o1_ref), (x2_ref, o2_ref), (x3_ref, o3_ref)):
        hp = orf.shape[1]
        acc = jnp.zeros(orf.shape, jnp.float32)
        for dh in range(3):
            for dw in range(3):
                acc = acc + xr[:, dh:dh + hp, dw:dw + hp, :]
        orf[...] = acc * (1.0 / 9.0)


def _pool3(f1, f2, f3):
    pads = ((0, 0), (1, 1), (1, 1), (0, 0))
    return pl.pallas_call(
        _pool3_kernel,
        out_shape=[jax.ShapeDtypeStruct(f.shape, jnp.float32)
                   for f in (f1, f2, f3)],
    )(jnp.pad(f1, pads), jnp.pad(f2, pads), jnp.pad(f3, pads))


_P = _H * _W
_PAD = 32


def _nn_kernel(emb_ref, bank_ref, best_ref, pred_ref, e2_s, d2p_s, cand_s,
               mp_s, m2p_s):
    n = pl.program_id(0)
    emb = emb_ref[...]

    @pl.when(n == 0)
    def _():
        e2_s[...] = jnp.sum(emb * emb, axis=-1)

    mrow = bank_ref[0]
    mp_s[_PAD:_PAD + _P] = mrow
    m2 = jnp.sum(mrow * mrow, axis=-1)
    m2p_s[_PAD:_PAD + _P] = m2
    e2 = e2_s[...]

    pcol = jax.lax.broadcasted_iota(jnp.int32, (_B, _P), 1)
    hh = pcol // _W
    ww = pcol % _W

    acc = None
    dot_c = None
    for dh in (-1, 0, 1):
        for dw in (-1, 0, 1):
            s = dh * _W + dw
            ms = mp_s[_PAD + s:_PAD + s + _P]
            m2s = m2p_s[_PAD + s:_PAD + s + _P]
            dot_o = jnp.sum(emb * ms[None], axis=-1)
            cand2 = e2 + m2s[None] - 2.0 * dot_o
            if dh == 0 and dw == 0:
                dot_c = dot_o
            else:
                valid = ((hh + dh >= 0) & (hh + dh < _H)
                         & (ww + dw >= 0) & (ww + dw < _W))
                cand2 = jnp.where(valid, cand2, jnp.inf)
            acc = cand2 if acc is None else jnp.minimum(acc, cand2)
    cand_s[n] = acc

    dotsum = jnp.sum(dot_c, axis=1)
    d2p_s[n] = jnp.sum(m2) - 2.0 * dotsum

    @pl.when(n == _N_BANK - 1)
    def _():
        e2sum = jnp.sum(e2_s[...], axis=1)
        dc = jnp.maximum(d2p_s[...] + e2sum[None, :], 1e-12)
        a = dc[:, None, :]
        b2 = dc[None, :, :]
        im = jax.lax.broadcasted_iota(jnp.int32, (_N_BANK, _N_BANK, 1), 0)
        inn = jax.lax.broadcasted_iota(jnp.int32, (_N_BANK, _N_BANK, 1), 1)
        lt = (a < b2) | ((a == b2) & (im < inn))
        rank = jnp.sum(lt.astype(jnp.int32), axis=0)
        sel = rank < _K_IM
        d_im = jnp.sqrt(dc)
        pred_ref[...] = jnp.sum(jnp.where(sel, d_im, 0.0), axis=0,
                                keepdims=True).T * (1.0 / _K_IM)
        pen = jnp.where(sel, 0.0, 1e30)
        bm = jnp.min(cand_s[...] + pen[:, :, None], axis=0)
        best_ref[...] = jnp.sqrt(jnp.maximum(bm, 1e-12))


def _nn_call(emb, bank):
    return pl.pallas_call(
        _nn_kernel,
        grid=(_N_BANK,),
        in_specs=[
            pl.BlockSpec((_B, _P, _C_EMB), lambda n: (0, 0, 0)),
            pl.BlockSpec((1, _P, _C_EMB), lambda n: (n, 0, 0)),
        ],
        out_specs=[
            pl.BlockSpec((_B, _P), lambda n: (0, 0)),
            pl.BlockSpec((_B, 1), lambda n: (0, 0)),
        ],
        out_shape=[
            jax.ShapeDtypeStruct((_B, _P), jnp.float32),
            jax.ShapeDtypeStruct((_B, 1), jnp.float32),
        ],
        scratch_shapes=[
            pltpu.VMEM((_B, _P), jnp.float32),
            pltpu.VMEM((_N_BANK, _B), jnp.float32),
            pltpu.VMEM((_N_BANK, _B, _P), jnp.float32),
            pltpu.VMEM((_P + 2 * _PAD, _C_EMB), jnp.float32),
            pltpu.VMEM((_P + 2 * _PAD,), jnp.float32),
        ],
    )(emb, bank)


def _amap_kernel(l_ref, x_ref, o_ref):
    lm = l_ref[...]
    for b in range(_B):
        t = jax.lax.dot_general(lm, x_ref[b], (((1,), (0,)), ((), ())),
                                precision=_HIGH,
                                preferred_element_type=jnp.float32)
        o_ref[b] = jax.lax.dot_general(t, lm, (((1,), (1,)), ((), ())),
                                       precision=_HIGH,
                                       preferred_element_type=jnp.float32)


def _amap_call(lmat, best):
    return pl.pallas_call(
        _amap_kernel,
        out_shape=jax.ShapeDtypeStruct((_B, _H_IMG, _H_IMG), jnp.float32),
    )(lmat, best)


def _gauss_matrix():
    size, sigma = 33, 4.0
    ax = np.arange(size, dtype=np.float64) - size // 2
    k = np.exp(-(ax ** 2) / (2.0 * sigma ** 2))
    k = (k / k.sum()).astype(np.float32)
    g = np.zeros((_H_IMG, _H_IMG), np.float32)
    for i in range(_H_IMG):
        for t in range(size):
            j = i + t - size // 2
            if 0 <= j < _H_IMG:
                g[i, j] += k[t]
    return g


_G_MAT = _gauss_matrix()


def kernel(input_tensor, W1, W2, W3, memory_bank):
    xp = jnp.pad(input_tensor, ((0, 0), (0, 0), (3, 3), (3, 3)))
    taps = [xp[:, :, kh:kh + 109:4, kw:kw + 109:4]
            for kh in range(7) for kw in range(7)]
    p1 = (jnp.stack(taps, 0).reshape(7, 7, _B, 3, _H, _W)
          .transpose(2, 4, 5, 3, 0, 1).reshape(_B * _H * _W, 147))
    f1 = _mm_relu(p1, W1.reshape(64, 147).T).reshape(_B, _H, _W, 64)

    f1p = jnp.pad(f1, ((0, 0), (1, 1), (1, 1), (0, 0)))
    taps2 = [f1p[:, kh:kh + 27:2, kw:kw + 27:2, :].transpose(0, 3, 1, 2)
             for kh in range(3) for kw in range(3)]
    p2 = (jnp.stack(taps2, 0).reshape(3, 3, _B, 64, 14, 14)
          .transpose(2, 4, 5, 3, 0, 1).reshape(_B * 196, 576))
    f2 = _mm_relu(p2, W2.reshape(128, 576).T).reshape(_B, 14, 14, 128)

    f2p = jnp.pad(f2, ((0, 0), (1, 1), (1, 1), (0, 0)))
    taps3 = [f2p[:, kh:kh + 13:2, kw:kw + 13:2, :].transpose(0, 3, 1, 2)
             for kh in range(3) for kw in range(3)]
    p3 = (jnp.stack(taps3, 0).reshape(3, 3, _B, 128, 7, 7)
          .transpose(2, 4, 5, 3, 0, 1).reshape(_B * 49, 1152))
    f3 = _mm_relu(p3, W3.reshape(256, 1152).T).reshape(_B, 7, 7, 256)

    g1, g2, g3 = _pool3(f1, f2, f3)
    g2u = jnp.repeat(jnp.repeat(g2, 2, axis=1), 2, axis=2)
    g3u = jnp.repeat(jnp.repeat(g3, 4, axis=1), 4, axis=2)
    emb = jnp.concatenate([g1, g2u, g3u], axis=-1)

    best, pred = _nn_call(emb.reshape(_B, _P, _C_EMB),
                          memory_bank.reshape(_N_BANK, _P, _C_EMB))
    best = best.reshape(_B, _H, _W)
    pred_score = pred.reshape(_B)

    rmat = jax.image.resize(jnp.eye(_H, dtype=jnp.float32), (_H_IMG, _H),
                            method='bilinear')
    lmat = jnp.matmul(jnp.asarray(_G_MAT), rmat, precision=_HIGH)
    amap = _amap_call(lmat, best).reshape(_B, 1, _H_IMG, _H_IMG)
    return pred_score, amap


# speedup vs baseline: 1.2740x; 1.2740x over previous
import numpy as np
import jax
import jax.numpy as jnp
from jax.experimental import pallas as pl
from jax.experimental.pallas import tpu as pltpu

_B = 4
_H_IMG = 112
_H = _W = 28
_C_EMB = 448
_N_BANK = 100
_K_IM = 50
_WIN = 3
_HIGH = jax.lax.Precision.HIGHEST


def _mm_relu_kernel(a_ref, w_ref, o_ref):
    o_ref[...] = jax.nn.relu(
        jax.lax.dot_general(a_ref[...], w_ref[...], (((1,), (0,)), ((), ())),
                            precision=_HIGH,
                            preferred_element_type=jnp.float32))


def _mm_relu(a, w):
    m, _ = a.shape
    _, n = w.shape
    return pl.pallas_call(
        _mm_relu_kernel,
        out_shape=jax.ShapeDtypeStruct((m, n), jnp.float32),
    )(a, w)


def _pool3_kernel(x1_ref, x2_ref, x3_ref, o1_ref, o2_ref, o3_ref):
    for xr, orf in ((x1_ref, o1_ref), (x2_ref, o2_ref), (x3_ref, o3_ref)):
        hp = orf.shape[1]
        acc = jnp.zeros(orf.shape, jnp.float32)
        for dh in range(3):
            for dw in range(3):
                acc = acc + xr[:, dh:dh + hp, dw:dw + hp, :]
        orf[...] = acc * (1.0 / 9.0)


def _pool3(f1, f2, f3):
    pads = ((0, 0), (1, 1), (1, 1), (0, 0))
    return pl.pallas_call(
        _pool3_kernel,
        out_shape=[jax.ShapeDtypeStruct(f.shape, jnp.float32)
                   for f in (f1, f2, f3)],
    )(jnp.pad(f1, pads), jnp.pad(f2, pads), jnp.pad(f3, pads))


def _nn_kernel(emb_ref, bank_ref, best_ref, pred_ref, e2_s, d2p_s, cand_s,
               acc_s, msh_s, m2sh_s):
    n = pl.program_id(0)
    emb = emb_ref[...]

    @pl.when(n == 0)
    def _():
        e2_s[...] = jnp.sum(emb * emb, axis=-1)

    mrow = bank_ref[0]
    m2 = jnp.sum(mrow * mrow, axis=-1)
    e2 = e2_s[...]

    msh_s[0] = mrow
    msh_s[1, :, 0:_W - 1] = mrow[:, 1:_W]
    msh_s[2, :, 1:_W] = mrow[:, 0:_W - 1]
    m2sh_s[0] = m2
    m2sh_s[1, :, 0:_W - 1] = m2[:, 1:_W]
    m2sh_s[2, :, 1:_W] = m2[:, 0:_W - 1]

    wcol = jax.lax.broadcasted_iota(jnp.int32, (_H, _W), 1)
    acc_s[...] = jnp.full((_B, _H, _W), jnp.inf, jnp.float32)
    dot_c = None
    for dh in (-1, 0, 1):
        h0, h1 = max(0, -dh), _H - max(0, dh)
        for dwi, dw in ((0, 0), (1, 1), (2, -1)):
            ms = msh_s[dwi, h0 + dh:h1 + dh]
            m2s = m2sh_s[dwi, h0 + dh:h1 + dh]
            dot_o = jnp.sum(emb[:, h0:h1] * ms[None], axis=-1)
            cand2 = e2[:, h0:h1] + m2s[None] - 2.0 * dot_o
            if dh == 0 and dw == 0:
                dot_c = dot_o
            if dw == 1:
                cand2 = jnp.where(wcol[h0:h1][None] < _W - 1, cand2,
                                  jnp.inf)
            elif dw == -1:
                cand2 = jnp.where(wcol[h0:h1][None] >= 1, cand2, jnp.inf)
            acc_s[:, h0:h1] = jnp.minimum(acc_s[:, h0:h1], cand2)
    cand_s[n] = acc_s[...]

    m2sum = jnp.sum(m2)
    dotsum = jnp.sum(dot_c, axis=(1, 2))
    d2p_s[n] = m2sum - 2.0 * dotsum

    @pl.when(n == _N_BANK - 1)
    def _():
        e2sum = jnp.sum(e2_s[...], axis=(1, 2))
        dc = jnp.maximum(d2p_s[...] + e2sum[None, :], 1e-12)
        a = dc[:, None, :]
        b2 = dc[None, :, :]
        im = jax.lax.broadcasted_iota(jnp.int32, (_N_BANK, _N_BANK, 1), 0)
        inn = jax.lax.broadcasted_iota(jnp.int32, (_N_BANK, _N_BANK, 1), 1)
        lt = (a < b2) | ((a == b2) & (im < inn))
        rank = jnp.sum(lt.astype(jnp.int32), axis=0)
        sel = rank < _K_IM
        d_im = jnp.sqrt(dc)
        pred_ref[...] = jnp.sum(jnp.where(sel, d_im, 0.0), axis=0,
                                keepdims=True).T * (1.0 / _K_IM)
        pen = jnp.where(sel, 0.0, 1e30)
        bm = jnp.min(cand_s[...] + pen[:, :, None, None], axis=0)
        best_ref[...] = jnp.sqrt(jnp.maximum(bm, 1e-12))


def _nn_call(emb, bank):
    return pl.pallas_call(
        _nn_kernel,
        grid=(_N_BANK,),
        in_specs=[
            pl.BlockSpec((_B, _H, _W, _C_EMB), lambda n: (0, 0, 0, 0)),
            pl.BlockSpec((1, _H, _W, _C_EMB), lambda n: (n, 0, 0, 0)),
        ],
        out_specs=[
            pl.BlockSpec((_B, _H, _W), lambda n: (0, 0, 0)),
            pl.BlockSpec((_B, 1), lambda n: (0, 0)),
        ],
        out_shape=[
            jax.ShapeDtypeStruct((_B, _H, _W), jnp.float32),
            jax.ShapeDtypeStruct((_B, 1), jnp.float32),
        ],
        scratch_shapes=[
            pltpu.VMEM((_B, _H, _W), jnp.float32),
            pltpu.VMEM((_N_BANK, _B), jnp.float32),
            pltpu.VMEM((_N_BANK, _B, _H, _W), jnp.float32),
            pltpu.VMEM((_B, _H, _W), jnp.float32),
            pltpu.VMEM((3, _H, _W, _C_EMB), jnp.float32),
            pltpu.VMEM((3, _H, _W), jnp.float32),
        ],
    )(emb, bank)


def _amap_kernel(l_ref, x_ref, o_ref):
    lm = l_ref[...]
    for b in range(_B):
        t = jax.lax.dot_general(lm, x_ref[b], (((1,), (0,)), ((), ())),
                                precision=_HIGH,
                                preferred_element_type=jnp.float32)
        o_ref[b] = jax.lax.dot_general(t, lm, (((1,), (1,)), ((), ())),
                                       precision=_HIGH,
                                       preferred_element_type=jnp.float32)


def _amap_call(lmat, best):
    return pl.pallas_call(
        _amap_kernel,
        out_shape=jax.ShapeDtypeStruct((_B, _H_IMG, _H_IMG), jnp.float32),
    )(lmat, best)


def _gauss_matrix():
    size, sigma = 33, 4.0
    ax = np.arange(size, dtype=np.float64) - size // 2
    k = np.exp(-(ax ** 2) / (2.0 * sigma ** 2))
    k = (k / k.sum()).astype(np.float32)
    g = np.zeros((_H_IMG, _H_IMG), np.float32)
    for i in range(_H_IMG):
        for t in range(size):
            j = i + t - size // 2
            if 0 <= j < _H_IMG:
                g[i, j] += k[t]
    return g


_G_MAT = _gauss_matrix()


def kernel(input_tensor, W1, W2, W3, memory_bank):
    xp = jnp.pad(input_tensor, ((0, 0), (0, 0), (3, 3), (3, 3)))
    taps = [xp[:, :, kh:kh + 109:4, kw:kw + 109:4]
            for kh in range(7) for kw in range(7)]
    p1 = (jnp.stack(taps, 0).reshape(7, 7, _B, 3, _H, _W)
          .transpose(2, 4, 5, 3, 0, 1).reshape(_B * _H * _W, 147))
    f1 = _mm_relu(p1, W1.reshape(64, 147).T).reshape(_B, _H, _W, 64)

    f1p = jnp.pad(f1, ((0, 0), (1, 1), (1, 1), (0, 0)))
    taps2 = [f1p[:, kh:kh + 27:2, kw:kw + 27:2, :].transpose(0, 3, 1, 2)
             for kh in range(3) for kw in range(3)]
    p2 = (jnp.stack(taps2, 0).reshape(3, 3, _B, 64, 14, 14)
          .transpose(2, 4, 5, 3, 0, 1).reshape(_B * 196, 576))
    f2 = _mm_relu(p2, W2.reshape(128, 576).T).reshape(_B, 14, 14, 128)

    f2p = jnp.pad(f2, ((0, 0), (1, 1), (1, 1), (0, 0)))
    taps3 = [f2p[:, kh:kh + 13:2, kw:kw + 13:2, :].transpose(0, 3, 1, 2)
             for kh in range(3) for kw in range(3)]
    p3 = (jnp.stack(taps3, 0).reshape(3, 3, _B, 128, 7, 7)
          .transpose(2, 4, 5, 3, 0, 1).reshape(_B * 49, 1152))
    f3 = _mm_relu(p3, W3.reshape(256, 1152).T).reshape(_B, 7, 7, 256)

    g1, g2, g3 = _pool3(f1, f2, f3)
    g2u = jnp.repeat(jnp.repeat(g2, 2, axis=1), 2, axis=2)
    g3u = jnp.repeat(jnp.repeat(g3, 4, axis=1), 4, axis=2)
    emb = jnp.concatenate([g1, g2u, g3u], axis=-1)

    best, pred = _nn_call(emb, memory_bank)
    pred_score = pred.reshape(_B)

    rmat = jax.image.resize(jnp.eye(_H, dtype=jnp.float32), (_H_IMG, _H),
                            method='bilinear')
    lmat = jnp.matmul(jnp.asarray(_G_MAT), rmat, precision=_HIGH)
    amap = _amap_call(lmat, best).reshape(_B, 1, _H_IMG, _H_IMG)
    return pred_score, amap
